# baseline (speedup 1.0000x reference)
import functools

import jax
import jax.numpy as jnp
from jax import lax
from jax.experimental import pallas as pl
from jax.experimental.pallas import tpu as pltpu
from jax.experimental.pallas import tpu_sc as plsc

VOCAB = 1000000
SEQ_LEN = 2048
EMBED_DIM = 64
BATCH = 4
TOTAL = BATCH * SEQ_LEN
NUM_WORKERS = 32
ROWS_PER_W = TOTAL // NUM_WORKERS
ELS_PER_W = ROWS_PER_W * EMBED_DIM
LANES = 16


def _body(eidx_hbm, tok_hbm, pos_hbm, out_hbm, idx_v, rows_v, pos_v, sem):
    wid = lax.axis_index("s") * 2 + lax.axis_index("c")
    ebase = wid * ELS_PER_W
    pos_ebase = lax.rem(ebase, SEQ_LEN * EMBED_DIM)

    pltpu.sync_copy(eidx_hbm.at[pl.ds(ebase, ELS_PER_W)], idx_v)
    pltpu.sync_copy(pos_hbm.at[pl.ds(pos_ebase, ELS_PER_W)], pos_v)

    pltpu.async_copy(tok_hbm.at[idx_v], rows_v, sem).wait()

    def vec_fn(i, carry):
        sl = pl.ds(i * LANES, LANES)
        rows_v[sl] = rows_v[sl] + pos_v[sl]
        return carry

    lax.fori_loop(0, ELS_PER_W // LANES, vec_fn, 0)

    pltpu.sync_copy(rows_v, out_hbm.at[pl.ds(ebase, ELS_PER_W)])


@jax.jit
def _run(eidx, tok_flat, pos_flat):
    mesh = plsc.VectorSubcoreMesh(core_axis_name="c", subcore_axis_name="s")
    f = functools.partial(
        pl.kernel,
        out_type=jax.ShapeDtypeStruct((TOTAL * EMBED_DIM,), jnp.float32),
        mesh=mesh,
        scratch_types=[
            pltpu.VMEM((ELS_PER_W,), jnp.int32),
            pltpu.VMEM((ELS_PER_W,), jnp.float32),
            pltpu.VMEM((ELS_PER_W,), jnp.float32),
            pltpu.SemaphoreType.DMA,
        ],
        compiler_params=pltpu.CompilerParams(use_tc_tiling_on_sc=False),
    )(_body)
    return f(eidx, tok_flat, pos_flat)


def kernel(inputs, token_table, position_table):
    flat = inputs.astype(jnp.int32).reshape(TOTAL)
    eidx = (flat[:, None] * EMBED_DIM
            + jnp.arange(EMBED_DIM, dtype=jnp.int32)[None, :]).reshape(-1)
    tok_flat = token_table.reshape(VOCAB * EMBED_DIM)
    pos_flat = position_table.reshape(SEQ_LEN * EMBED_DIM)
    out = _run(eidx, tok_flat, pos_flat)
    return out.reshape(BATCH, SEQ_LEN, EMBED_DIM)


# speedup vs baseline: 2.6484x; 2.6484x over previous
import functools

import jax
import jax.numpy as jnp
from jax import lax
from jax.experimental import pallas as pl
from jax.experimental.pallas import tpu as pltpu
from jax.experimental.pallas import tpu_sc as plsc

VOCAB = 1000000
SEQ_LEN = 2048
EMBED_DIM = 64
BATCH = 4
TOTAL = BATCH * SEQ_LEN
NUM_WORKERS = 32
ROWS_PER_W = TOTAL // NUM_WORKERS
LANES = 16
N_GROUPS = ROWS_PER_W // LANES
BLK = 16


def _body(idx_hbm, tok_hbm, pos_hbm, out_hbm, idx_v, rows_v, pos_v, *sems):
    wid = lax.axis_index("s") * 2 + lax.axis_index("c")
    base = wid * ROWS_PER_W
    pos_base = lax.rem(base, SEQ_LEN)

    pltpu.sync_copy(idx_hbm.at[pl.ds(base, ROWS_PER_W)], idx_v)
    pltpu.sync_copy(pos_hbm.at[pl.ds(pos_base, ROWS_PER_W)], pos_v)

    for g in range(N_GROUPS):
        j0 = g * LANES
        idx16 = idx_v[pl.ds(j0, LANES)]
        for jj in range(LANES):
            i = idx16[jj]
            t = lax.shift_right_logical(i, 4)
            r = lax.bitwise_and(i, BLK - 1)
            pltpu.async_copy(tok_hbm.at[t, r], rows_v.at[j0 + jj], sems[g])

    for g in range(N_GROUPS):
        pltpu.make_async_copy(
            pos_hbm.at[pl.ds(0, LANES)],
            rows_v.at[pl.ds(g * LANES, LANES)], sems[g]
        ).wait()

    def grp_fn(g, carry):
        j0 = g * LANES
        for jj in range(LANES):
            for c in range(EMBED_DIM // LANES):
                sl = pl.ds(c * LANES, LANES)
                rows_v[j0 + jj, sl] = rows_v[j0 + jj, sl] + pos_v[j0 + jj, sl]
        return carry

    lax.fori_loop(0, N_GROUPS, grp_fn, 0)

    pltpu.sync_copy(rows_v, out_hbm.at[pl.ds(base, ROWS_PER_W)])


@jax.jit
def _run(idx, tok3, position_table):
    mesh = plsc.VectorSubcoreMesh(core_axis_name="c", subcore_axis_name="s")
    f = functools.partial(
        pl.kernel,
        out_type=jax.ShapeDtypeStruct((TOTAL, EMBED_DIM), jnp.float32),
        mesh=mesh,
        scratch_types=[
            pltpu.VMEM((ROWS_PER_W,), jnp.int32),
            pltpu.VMEM((ROWS_PER_W, EMBED_DIM), jnp.float32),
            pltpu.VMEM((ROWS_PER_W, EMBED_DIM), jnp.float32),
        ] + [pltpu.SemaphoreType.DMA] * N_GROUPS,
    )(_body)
    return f(idx, tok3, position_table)


def kernel(inputs, token_table, position_table):
    idx = inputs.astype(jnp.int32).reshape(TOTAL)
    tok3 = token_table.reshape(VOCAB // BLK, BLK, EMBED_DIM)
    out = _run(idx, tok3, position_table)
    return out.reshape(BATCH, SEQ_LEN, EMBED_DIM)
